# baseline (speedup 1.0000x reference)
import functools

import jax
import jax.numpy as jnp
from jax import lax
from jax.experimental import pallas as pl
from jax.experimental.pallas import tpu as pltpu
from jax.experimental.pallas import tpu_sc as plsc

N_NODES = 10000
D = 128
N_EDGES = 320000

NC = 2
NS = 16
NW = NC * NS

CHUNK = 96
NBUF = 3
CPT0 = 105
CPT1 = 105
E_PAD = NS * CHUNK * (CPT0 + CPT1)
IDXBUF = 48


def _phases(cpt):
    out = []
    off = 0
    while off < cpt:
        n = min(IDXBUF, cpt - off)
        out.append((off, n))
        off += n
    return out

ACC_ROWS = 10112
ZROWS = ACC_ROWS // NS

ROW_BLOCK = 400
TC_GRID = N_NODES // ROW_BLOCK


def _spmm_partials(h, srcA, dstA, srcB, dstB, zeros_hbm):
    mesh = plsc.VectorSubcoreMesh(core_axis_name="c", subcore_axis_name="s")

    @functools.partial(
        pl.kernel,
        mesh=mesh,
        out_type=jax.ShapeDtypeStruct((NC, ACC_ROWS, D), jnp.float32),
        scratch_types=[
            pltpu.VMEM((IDXBUF, CHUNK), jnp.int32),
            pltpu.VMEM((IDXBUF, CHUNK), jnp.int32),
            pltpu.VMEM((NBUF, CHUNK, D), jnp.float32),
            pltpu.VMEM_SHARED((ACC_ROWS, D), jnp.float32),
            pltpu.SemaphoreType.DMA,
            pltpu.SemaphoreType.DMA,
        ],
    )
    def k(h_hbm, srcA_hbm, dstA_hbm, srcB_hbm, dstB_hbm, z_hbm, out_hbm,
          src_v, dst_v, rows_v, acc_sh, gsem, ssem):
        cid = lax.axis_index("c")
        sid = lax.axis_index("s")

        pltpu.sync_copy(z_hbm, acc_sh.at[pl.ds(sid * ZROWS, ZROWS)])
        plsc.subcore_barrier()

        def start_g(j, b):
            pltpu.async_copy(h_hbm.at[src_v.at[j]], rows_v.at[b], gsem)

        def wait_g(j, b):
            pltpu.make_async_copy(h_hbm.at[src_v.at[j]], rows_v.at[b], gsem).wait()

        def start_s(j, b):
            pltpu.async_copy(rows_v.at[b], acc_sh.at[dst_v.at[j]], ssem, add=True)

        def wait_s(j, b):
            pltpu.make_async_copy(rows_v.at[b], acc_sh.at[dst_v.at[j]], ssem).wait()

        def phase(src_hbm, dst_hbm, off, n):
            pltpu.sync_copy(src_hbm.at[sid].at[pl.ds(off, n)],
                            src_v.at[pl.ds(0, n)])
            pltpu.sync_copy(dst_hbm.at[sid].at[pl.ds(off, n)],
                            dst_v.at[pl.ds(0, n)])

            start_g(0, 0)
            start_g(1, 1)
            wait_g(0, 0)
            start_s(0, 0)
            start_g(2, 2)

            def body(j, carry):
                b = j % NBUF
                wait_g(j, b)
                start_s(j, b)
                wait_s(j - 1, (j - 1) % NBUF)
                start_g(j + 2, (j + 2) % NBUF)
                return carry

            lax.fori_loop(1, n - 2, body, 0)

            j = n - 2
            wait_g(j, j % NBUF)
            start_s(j, j % NBUF)
            wait_s(j - 1, (j - 1) % NBUF)
            j = n - 1
            wait_g(j, j % NBUF)
            start_s(j, j % NBUF)
            wait_s(j - 1, (j - 1) % NBUF)
            wait_s(j, j % NBUF)

        @pl.when(cid == 0)
        def _():
            for off, n in _phases(CPT0):
                phase(srcA_hbm, dstA_hbm, off, n)

        @pl.when(cid == 1)
        def _():
            for off, n in _phases(CPT1):
                phase(srcB_hbm, dstB_hbm, off, n)

        plsc.subcore_barrier()
        pltpu.sync_copy(
            acc_sh.at[pl.ds(sid * ZROWS, ZROWS)],
            out_hbm.at[cid].at[pl.ds(sid * ZROWS, ZROWS)],
        )

    return k(h, srcA, dstA, srcB, dstB, zeros_hbm)


def _tc_layer1(p0, p1, W1, b1):
    def body(p0_ref, p1_ref, w_ref, b_ref, o_ref):
        s = p0_ref[...] + p1_ref[...]
        t = jnp.dot(s, w_ref[...], preferred_element_type=jnp.float32)
        o_ref[...] = jnp.maximum(t + b_ref[...], 0.0)

    return pl.pallas_call(
        body,
        grid=(TC_GRID,),
        in_specs=[
            pl.BlockSpec((ROW_BLOCK, D), lambda i: (i, 0)),
            pl.BlockSpec((ROW_BLOCK, D), lambda i: (i, 0)),
            pl.BlockSpec((D, D), lambda i: (0, 0)),
            pl.BlockSpec((1, D), lambda i: (0, 0)),
        ],
        out_specs=pl.BlockSpec((ROW_BLOCK, D), lambda i: (i, 0)),
        out_shape=jax.ShapeDtypeStruct((N_NODES, D), jnp.float32),
    )(p0, p1, W1, b1.reshape(1, D))


def _tc_layer2(q0, q1, W2, b2):
    def body(q0_ref, q1_ref, w_ref, b_ref, o_ref):
        s = q0_ref[...] + q1_ref[...]
        z = jnp.dot(s, w_ref[...], preferred_element_type=jnp.float32)
        z = z + b_ref[...]
        m = jnp.max(z, axis=1, keepdims=True)
        e = z - m
        lse = jnp.log(jnp.sum(jnp.exp(e), axis=1, keepdims=True))
        o_ref[...] = e - lse

    return pl.pallas_call(
        body,
        grid=(TC_GRID,),
        in_specs=[
            pl.BlockSpec((ROW_BLOCK, D), lambda i: (i, 0)),
            pl.BlockSpec((ROW_BLOCK, D), lambda i: (i, 0)),
            pl.BlockSpec((D, D), lambda i: (0, 0)),
            pl.BlockSpec((1, D), lambda i: (0, 0)),
        ],
        out_specs=pl.BlockSpec((ROW_BLOCK, D), lambda i: (i, 0)),
        out_shape=jax.ShapeDtypeStruct((N_NODES, D), jnp.float32),
    )(q0, q1, W2, b2.reshape(1, D))


def kernel(x, edge_index, W1, b1, W2, b2):
    src = edge_index[0].astype(jnp.int32)
    dst = edge_index[1].astype(jnp.int32)
    pad = E_PAD - N_EDGES
    src = jnp.concatenate([src, jnp.zeros((pad,), jnp.int32)])
    dst = jnp.concatenate([dst, jnp.full((pad,), N_NODES, jnp.int32)])
    ea = NS * CPT0 * CHUNK
    srcA = src[:ea].reshape(NS, CPT0, CHUNK)
    dstA = dst[:ea].reshape(NS, CPT0, CHUNK)
    srcB = src[ea:].reshape(NS, CPT1, CHUNK)
    dstB = dst[ea:].reshape(NS, CPT1, CHUNK)
    zeros_hbm = jnp.zeros((ZROWS, D), jnp.float32)

    P = _spmm_partials(x, srcA, dstA, srcB, dstB, zeros_hbm)
    h = _tc_layer1(P[0], P[1], W1, b1)
    Q = _spmm_partials(h, srcA, dstA, srcB, dstB, zeros_hbm)
    return _tc_layer2(Q[0], Q[1], W2, b2)


# speedup vs baseline: 1.1571x; 1.1571x over previous
import functools

import jax
import jax.numpy as jnp
from jax import lax
from jax.experimental import pallas as pl
from jax.experimental.pallas import tpu as pltpu
from jax.experimental.pallas import tpu_sc as plsc

N_NODES = 10000
D = 128
N_EDGES = 320000

NC = 2
NS = 16
NW = NC * NS

CHUNK = 96
NBUF = 3
CPT0 = 147
CPT1 = 63
E_PAD = NS * CHUNK * (CPT0 + CPT1)
IDXBUF = 48


def _phases(cpt):
    out = []
    off = 0
    while off < cpt:
        n = min(IDXBUF, cpt - off)
        out.append((off, n))
        off += n
    return out

ACC_ROWS = 10112
ZROWS = ACC_ROWS // NS

ROW_BLOCK = 400
TC_GRID = N_NODES // ROW_BLOCK


def _spmm_partials(h, srcA, dstA, srcB, dstB, zeros_hbm):
    mesh = plsc.VectorSubcoreMesh(core_axis_name="c", subcore_axis_name="s")

    @functools.partial(
        pl.kernel,
        mesh=mesh,
        out_type=jax.ShapeDtypeStruct((NC, ACC_ROWS, D), jnp.float32),
        scratch_types=[
            pltpu.VMEM((IDXBUF, CHUNK), jnp.int32),
            pltpu.VMEM((IDXBUF, CHUNK), jnp.int32),
            pltpu.VMEM((NBUF, CHUNK, D), jnp.float32),
            pltpu.VMEM_SHARED((ACC_ROWS, D), jnp.float32),
            pltpu.SemaphoreType.DMA,
            pltpu.SemaphoreType.DMA,
        ],
    )
    def k(h_hbm, srcA_hbm, dstA_hbm, srcB_hbm, dstB_hbm, z_hbm, out_hbm,
          src_v, dst_v, rows_v, acc_sh, gsem, ssem):
        cid = lax.axis_index("c")
        sid = lax.axis_index("s")

        pltpu.sync_copy(z_hbm, acc_sh.at[pl.ds(sid * ZROWS, ZROWS)])
        plsc.subcore_barrier()

        def start_g(j, b):
            pltpu.async_copy(h_hbm.at[src_v.at[j]], rows_v.at[b], gsem)

        def wait_g(j, b):
            pltpu.make_async_copy(h_hbm.at[src_v.at[j]], rows_v.at[b], gsem).wait()

        def start_s(j, b):
            pltpu.async_copy(rows_v.at[b], acc_sh.at[dst_v.at[j]], ssem, add=True)

        def wait_s(j, b):
            pltpu.make_async_copy(rows_v.at[b], acc_sh.at[dst_v.at[j]], ssem).wait()

        def phase(src_hbm, dst_hbm, off, n):
            pltpu.sync_copy(src_hbm.at[sid].at[pl.ds(off, n)],
                            src_v.at[pl.ds(0, n)])
            pltpu.sync_copy(dst_hbm.at[sid].at[pl.ds(off, n)],
                            dst_v.at[pl.ds(0, n)])

            start_g(0, 0)
            start_g(1, 1)
            wait_g(0, 0)
            start_s(0, 0)
            start_g(2, 2)

            def body(j, carry):
                b = j % NBUF
                wait_g(j, b)
                start_s(j, b)
                wait_s(j - 1, (j - 1) % NBUF)
                start_g(j + 2, (j + 2) % NBUF)
                return carry

            lax.fori_loop(1, n - 2, body, 0)

            j = n - 2
            wait_g(j, j % NBUF)
            start_s(j, j % NBUF)
            wait_s(j - 1, (j - 1) % NBUF)
            j = n - 1
            wait_g(j, j % NBUF)
            start_s(j, j % NBUF)
            wait_s(j - 1, (j - 1) % NBUF)
            wait_s(j, j % NBUF)

        @pl.when(cid == 0)
        def _():
            for off, n in _phases(CPT0):
                phase(srcA_hbm, dstA_hbm, off, n)

        @pl.when(cid == 1)
        def _():
            for off, n in _phases(CPT1):
                phase(srcB_hbm, dstB_hbm, off, n)

        plsc.subcore_barrier()
        pltpu.sync_copy(
            acc_sh.at[pl.ds(sid * ZROWS, ZROWS)],
            out_hbm.at[cid].at[pl.ds(sid * ZROWS, ZROWS)],
        )

    return k(h, srcA, dstA, srcB, dstB, zeros_hbm)


_PARTIAL_SPECS = [
    pl.BlockSpec((1, ROW_BLOCK, D), lambda i: (0, i, 0)),
    pl.BlockSpec((1, ROW_BLOCK, D), lambda i: (1, i, 0)),
    pl.BlockSpec((D, D), lambda i: (0, 0)),
    pl.BlockSpec((1, D), lambda i: (0, 0)),
]


def _tc_layer1(P, W1, b1):
    def body(p0_ref, p1_ref, w_ref, b_ref, o_ref):
        s = p0_ref[0] + p1_ref[0]
        t = jnp.dot(s, w_ref[...], preferred_element_type=jnp.float32)
        o_ref[...] = jnp.maximum(t + b_ref[...], 0.0)

    return pl.pallas_call(
        body,
        grid=(TC_GRID,),
        in_specs=_PARTIAL_SPECS,
        out_specs=pl.BlockSpec((ROW_BLOCK, D), lambda i: (i, 0)),
        out_shape=jax.ShapeDtypeStruct((N_NODES, D), jnp.float32),
    )(P, P, W1, b1.reshape(1, D))


def _tc_layer2(Q, W2, b2):
    def body(q0_ref, q1_ref, w_ref, b_ref, o_ref):
        s = q0_ref[0] + q1_ref[0]
        z = jnp.dot(s, w_ref[...], preferred_element_type=jnp.float32)
        z = z + b_ref[...]
        m = jnp.max(z, axis=1, keepdims=True)
        e = z - m
        lse = jnp.log(jnp.sum(jnp.exp(e), axis=1, keepdims=True))
        o_ref[...] = e - lse

    return pl.pallas_call(
        body,
        grid=(TC_GRID,),
        in_specs=_PARTIAL_SPECS,
        out_specs=pl.BlockSpec((ROW_BLOCK, D), lambda i: (i, 0)),
        out_shape=jax.ShapeDtypeStruct((N_NODES, D), jnp.float32),
    )(Q, Q, W2, b2.reshape(1, D))


def kernel(x, edge_index, W1, b1, W2, b2):
    src = edge_index[0].astype(jnp.int32)
    dst = edge_index[1].astype(jnp.int32)
    pad = E_PAD - N_EDGES
    src = jnp.concatenate([src, jnp.zeros((pad,), jnp.int32)])
    dst = jnp.concatenate([dst, jnp.full((pad,), N_NODES, jnp.int32)])
    ea = NS * CPT0 * CHUNK
    srcA = src[:ea].reshape(NS, CPT0, CHUNK)
    dstA = dst[:ea].reshape(NS, CPT0, CHUNK)
    srcB = src[ea:].reshape(NS, CPT1, CHUNK)
    dstB = dst[ea:].reshape(NS, CPT1, CHUNK)
    zeros_hbm = jnp.zeros((ZROWS, D), jnp.float32)

    P = _spmm_partials(x, srcA, dstA, srcB, dstB, zeros_hbm)
    h = _tc_layer1(P, W1, b1)
    Q = _spmm_partials(h, srcA, dstA, srcB, dstB, zeros_hbm)
    return _tc_layer2(Q, W2, b2)
